# Initial kernel scaffold; baseline (speedup 1.0000x reference)
import jax
import jax.numpy as jnp
from jax.experimental import pallas as pl


def kernel(encoder_features, word_ids, Wt1, bt1, Wt2, bt2, Ww1, bw1, Ww2, bw2):
    raise NotImplementedError("write your pallas kernel here")


# speedup vs baseline: 9.1679x; 9.1679x over previous
import numpy as np
import jax
import jax.numpy as jnp
from jax.experimental import pallas as pl
from jax.experimental.pallas import tpu as pltpu

_N = 32
_P = _N * (_N - 1) // 2
_R = 512
_H = 512
_V = 10000
_VT = 1280
_NT = 8
_NEG = -1.0e30


def _fidx(i, j):
    return (2 * _N - i - 1) * i // 2 + j - i - 1


def _build_consts():
    gi = np.zeros((_R, _N), np.float32)
    gj = np.zeros((_R, _N), np.float32)
    gw = np.zeros((_R, _N), np.float32)
    p = 0
    for i in range(_N):
        for j in range(i + 1, _N):
            gi[p, i] = 1.0
            gj[p, j] = 1.0
            if j <= _N - 2:
                gw[p, j + 1] = 1.0
            p += 1
    gi[_P, 0] = 1.0
    gj[_P, 0] = 1.0
    gw[_P, 1] = 1.0
    rj = np.zeros((_R, _N), np.float32)
    rq = np.zeros((_R, _N), np.float32)
    for j in range(1, _N):
        for q in range(j):
            r = j * (j - 1) // 2 + q
            rj[r, j] = 1.0
            rq[r, q] = 1.0
    return gi, gj, gw, gi.T.copy(), rj.T.copy(), rq


_GI, _GJ, _GW, _GIT, _RJT, _RQ = _build_consts()


def _mlp_kernel(enc, wt1, bt1, wt2, bt2, ww1, bw1, wids, gi, gj, gw,
                ww2, bw2, lr_out, rp_out, delta_out,
                hw_s, m_s, s_s, tl_s):
    t = pl.program_id(0)

    @pl.when(t == 0)
    def _prologue():
        e = enc[...]
        gim = gi[...]
        gjm = gj[...]
        at = jnp.dot(e, wt1[0:_H, :], preferred_element_type=jnp.float32)
        bt = jnp.dot(e, wt1[_H:2 * _H, :], preferred_element_type=jnp.float32)
        ht = jnp.maximum(
            jnp.dot(gim, at, preferred_element_type=jnp.float32)
            + jnp.dot(gjm, bt, preferred_element_type=jnp.float32)
            + bt1[...], 0.0)
        zt = jnp.dot(ht, wt2[...], preferred_element_type=jnp.float32) + bt2[...]
        lr_out[...] = -(jnp.maximum(-zt, 0.0) + jnp.log(1.0 + jnp.exp(-jnp.abs(zt))))
        rp_out[...] = -(jnp.maximum(zt, 0.0) + jnp.log(1.0 + jnp.exp(-jnp.abs(zt))))
        aw = jnp.dot(e, ww1[0:_H, :], preferred_element_type=jnp.float32)
        bw = jnp.dot(e, ww1[_H:2 * _H, :], preferred_element_type=jnp.float32)
        hw_s[...] = jnp.maximum(
            jnp.dot(gim, aw, preferred_element_type=jnp.float32)
            + jnp.dot(gjm, bw, preferred_element_type=jnp.float32)
            + bw1[...], 0.0)
        m_s[...] = jnp.full((_R, 1), _NEG, jnp.float32)
        s_s[...] = jnp.zeros((_R, 1), jnp.float32)
        tl_s[...] = jnp.zeros((_R, 1), jnp.float32)

    logits = jnp.dot(hw_s[...], ww2[...],
                     preferred_element_type=jnp.float32) + bw2[0]
    mt = jnp.max(logits, axis=1, keepdims=True)
    mn = jnp.maximum(m_s[...], mt)
    s_s[...] = (s_s[...] * jnp.exp(m_s[...] - mn)
                + jnp.sum(jnp.exp(logits - mn), axis=1, keepdims=True))
    m_s[...] = mn
    wid = jnp.dot(gw[...], wids[...],
                  preferred_element_type=jnp.float32).astype(jnp.int32)
    col = jax.lax.broadcasted_iota(jnp.int32, (_R, _VT), 1) + t * _VT
    tl_s[...] = tl_s[...] + jnp.sum(
        jnp.where(col == wid, logits, 0.0), axis=1, keepdims=True)

    @pl.when(t == _NT - 1)
    def _epilogue():
        delta_out[...] = tl_s[...] - (jnp.log(s_s[...]) + m_s[...])


def _dp_kernel(lr_in, rp_in, delta_in, git, gjm, rjt, rqm, out):
    rpmat = jnp.dot(git[...], rp_in[...] * gjm[...],
                    preferred_element_type=jnp.float32)
    dpmat = jnp.dot(git[...], delta_in[...] * gjm[...],
                    preferred_element_type=jnp.float32)
    lrmat = jnp.dot(rjt[...], lr_in[...] * rqm[...],
                    preferred_element_type=jnp.float32)

    ia0 = jax.lax.broadcasted_iota(jnp.int32, (_N, _N, _N), 0)
    ia1 = jax.lax.broadcasted_iota(jnp.int32, (_N, _N, _N), 1)
    ia2 = jax.lax.broadcasted_iota(jnp.int32, (_N, _N, _N), 2)
    i2 = jax.lax.broadcasted_iota(jnp.int32, (_N, _N), 0)
    p2 = jax.lax.broadcasted_iota(jnp.int32, (_N, _N), 1)

    vmat = dpmat + jnp.where(p2 > i2, rpmat, 0.0)
    maskd = (ia2 == ia1 + 1) & (((ia1 > ia0) & (ia1 <= _N - 2))
                                | ((ia0 == 0) & (ia1 == 0)))
    table = jnp.where(maskd, vmat[:, :, None], _NEG)

    def body(gap, tbl):
        tj = jnp.sum(jnp.where(ia2 == ia0 + gap, tbl, 0.0), axis=2)
        lrg = jnp.sum(jnp.where(ia1 == ia0 + gap, lrmat[None, :, :], 0.0),
                      axis=1)
        b = tj + lrg
        valid = (p2 >= i2 + 1) & (p2 <= i2 + gap - 1) & (i2 + gap <= _N - 1)
        b = jnp.where(valid, b, _NEG)
        scores = tbl + b[None, :, :]
        mx = jnp.max(scores, axis=2)
        new = jnp.log(jnp.sum(jnp.exp(scores - mx[:, :, None]), axis=2)) + mx
        wm = (ia2 == ia1 + gap) & ((ia0 < ia1) | ((ia0 == 0) & (ia1 == 0)))
        return jnp.where(wm, new[:, :, None], tbl)

    table = jax.lax.fori_loop(2, _N, body, table)
    out[...] = jnp.full((1, 1), 0.0) + jnp.sum(
        jnp.where((ia0 == 0) & (ia1 == 0) & (ia2 == _N - 1), table, 0.0))


def kernel(encoder_features, word_ids, Wt1, bt1, Wt2, bt2, Ww1, bw1, Ww2, bw2):
    f32 = jnp.float32
    ww2p = jnp.pad(Ww2, ((0, 0), (0, _NT * _VT - _V)))
    bw2p = jnp.pad(bw2, (0, _NT * _VT - _V),
                   constant_values=_NEG).reshape(_NT, 1, _VT)
    wids = word_ids.astype(f32).reshape(_N, 1)

    full = lambda shp: pl.BlockSpec(shp, lambda t: tuple(0 for _ in shp))
    lr, rp, delta = pl.pallas_call(
        _mlp_kernel,
        grid=(_NT,),
        in_specs=[
            full((_N, _H)),
            full((2 * _H, _H)),
            full((1, _H)),
            full((_H, 1)),
            full((1, 1)),
            full((2 * _H, _H)),
            full((1, _H)),
            full((_N, 1)),
            full((_R, _N)),
            full((_R, _N)),
            full((_R, _N)),
            pl.BlockSpec((_H, _VT), lambda t: (0, t)),
            pl.BlockSpec((1, 1, _VT), lambda t: (t, 0, 0)),
        ],
        out_specs=[full((_R, 1)), full((_R, 1)), full((_R, 1))],
        out_shape=[jax.ShapeDtypeStruct((_R, 1), f32)] * 3,
        scratch_shapes=[
            pltpu.VMEM((_R, _H), f32),
            pltpu.VMEM((_R, 1), f32),
            pltpu.VMEM((_R, 1), f32),
            pltpu.VMEM((_R, 1), f32),
        ],
    )(encoder_features, Wt1, bt1.reshape(1, _H), Wt2, bt2.reshape(1, 1),
      Ww1, bw1.reshape(1, _H), wids,
      jnp.asarray(_GI), jnp.asarray(_GJ), jnp.asarray(_GW), ww2p, bw2p)

    outv = pl.pallas_call(
        _dp_kernel,
        out_shape=jax.ShapeDtypeStruct((1, 1), f32),
    )(lr, rp, delta, jnp.asarray(_GIT), jnp.asarray(_GJ),
      jnp.asarray(_RJT), jnp.asarray(_RQ))
    return outv.reshape(())


# speedup vs baseline: 12.3527x; 1.3474x over previous
import numpy as np
import jax
import jax.numpy as jnp
from jax.experimental import pallas as pl
from jax.experimental.pallas import tpu as pltpu

_N = 32
_P = _N * (_N - 1) // 2
_R = 512
_H = 512
_V = 10000
_VT = 1280
_NT = 8
_NEG = -1.0e30


def _fidx(i, j):
    return (2 * _N - i - 1) * i // 2 + j - i - 1


def _build_consts():
    gi = np.zeros((_R, _N), np.float32)
    gj = np.zeros((_R, _N), np.float32)
    gw = np.zeros((_R, _N), np.float32)
    p = 0
    for i in range(_N):
        for j in range(i + 1, _N):
            gi[p, i] = 1.0
            gj[p, j] = 1.0
            if j <= _N - 2:
                gw[p, j + 1] = 1.0
            p += 1
    gi[_P, 0] = 1.0
    gj[_P, 0] = 1.0
    gw[_P, 1] = 1.0
    rj = np.zeros((_R, _N), np.float32)
    rq = np.zeros((_R, _N), np.float32)
    for j in range(1, _N):
        for q in range(j):
            r = j * (j - 1) // 2 + q
            rj[r, j] = 1.0
            rq[r, q] = 1.0
    return gi, gj, gw, gi.T.copy(), rj.T.copy(), rq


_GI, _GJ, _GW, _GIT, _RJT, _RQ = _build_consts()


def _mlp_kernel(enc, wt1, bt1, wt2, bt2, ww1, bw1, wids, gi, gj, gw,
                ww2, bw2, lr_out, rp_out, delta_out,
                hw_s, m_s, s_s, tl_s):
    t = pl.program_id(0)

    @pl.when(t == 0)
    def _prologue():
        e = enc[...]
        gim = gi[...]
        gjm = gj[...]
        at = jnp.dot(e, wt1[0:_H, :], preferred_element_type=jnp.float32)
        bt = jnp.dot(e, wt1[_H:2 * _H, :], preferred_element_type=jnp.float32)
        ht = jnp.maximum(
            jnp.dot(gim, at, preferred_element_type=jnp.float32)
            + jnp.dot(gjm, bt, preferred_element_type=jnp.float32)
            + bt1[...], 0.0)
        zt = jnp.dot(ht, wt2[...], preferred_element_type=jnp.float32) + bt2[...]
        lr_out[...] = -(jnp.maximum(-zt, 0.0) + jnp.log(1.0 + jnp.exp(-jnp.abs(zt))))
        rp_out[...] = -(jnp.maximum(zt, 0.0) + jnp.log(1.0 + jnp.exp(-jnp.abs(zt))))
        aw = jnp.dot(e, ww1[0:_H, :], preferred_element_type=jnp.float32)
        bw = jnp.dot(e, ww1[_H:2 * _H, :], preferred_element_type=jnp.float32)
        hw_s[...] = jnp.maximum(
            jnp.dot(gim, aw, preferred_element_type=jnp.float32)
            + jnp.dot(gjm, bw, preferred_element_type=jnp.float32)
            + bw1[...], 0.0)
        m_s[...] = jnp.full((_R, 1), _NEG, jnp.float32)
        s_s[...] = jnp.zeros((_R, 1), jnp.float32)
        tl_s[...] = jnp.zeros((_R, 1), jnp.float32)

    wcol = jax.lax.broadcasted_iota(jnp.int32, (_R, _VT), 1) + t * _VT
    logits = jnp.dot(hw_s[...].astype(jnp.bfloat16),
                     ww2[...].astype(jnp.bfloat16),
                     preferred_element_type=jnp.float32) + bw2[0]
    logits = jnp.where(wcol < _V, logits, _NEG)
    mt = jnp.max(logits, axis=1, keepdims=True)
    mn = jnp.maximum(m_s[...], mt)
    s_s[...] = (s_s[...] * jnp.exp(m_s[...] - mn)
                + jnp.sum(jnp.exp(logits - mn), axis=1, keepdims=True))
    m_s[...] = mn
    wid = jnp.dot(gw[...], wids[...],
                  preferred_element_type=jnp.float32).astype(jnp.int32)
    tl_s[...] = tl_s[...] + jnp.sum(
        jnp.where(wcol == wid, logits, 0.0), axis=1, keepdims=True)

    @pl.when(t == _NT - 1)
    def _epilogue():
        delta_out[...] = tl_s[...] - (jnp.log(s_s[...]) + m_s[...])


def _dp_kernel(lr_in, rp_in, delta_in, git, gjm, rjt, rqm, out):
    rpmat = jnp.dot(git[...], rp_in[...] * gjm[...],
                    preferred_element_type=jnp.float32)
    dpmat = jnp.dot(git[...], delta_in[...] * gjm[...],
                    preferred_element_type=jnp.float32)
    lrmat = jnp.dot(rjt[...], lr_in[...] * rqm[...],
                    preferred_element_type=jnp.float32)

    ia0 = jax.lax.broadcasted_iota(jnp.int32, (_N, _N, _N), 0)
    ia1 = jax.lax.broadcasted_iota(jnp.int32, (_N, _N, _N), 1)
    ia2 = jax.lax.broadcasted_iota(jnp.int32, (_N, _N, _N), 2)
    i2 = jax.lax.broadcasted_iota(jnp.int32, (_N, _N), 0)
    p2 = jax.lax.broadcasted_iota(jnp.int32, (_N, _N), 1)

    vmat = dpmat + jnp.where(p2 > i2, rpmat, 0.0)
    maskd = (ia2 == ia1 + 1) & (((ia1 > ia0) & (ia1 <= _N - 2))
                                | ((ia0 == 0) & (ia1 == 0)))
    table = jnp.where(maskd, vmat[:, :, None], _NEG)

    def body(gap, tbl):
        tj = jnp.sum(jnp.where(ia2 == ia0 + gap, tbl, 0.0), axis=2)
        lrg = jnp.sum(jnp.where(ia1 == ia0 + gap, lrmat[None, :, :], 0.0),
                      axis=1)
        b = tj + lrg
        valid = (p2 >= i2 + 1) & (p2 <= i2 + gap - 1) & (i2 + gap <= _N - 1)
        b = jnp.where(valid, b, _NEG)
        scores = tbl + b[None, :, :]
        mx = jnp.max(scores, axis=2)
        new = jnp.log(jnp.sum(jnp.exp(scores - mx[:, :, None]), axis=2)) + mx
        wm = (ia2 == ia1 + gap) & ((ia0 < ia1) | ((ia0 == 0) & (ia1 == 0)))
        return jnp.where(wm, new[:, :, None], tbl)

    table = jax.lax.fori_loop(2, _N, body, table)
    out[...] = jnp.full((1, 1), 0.0) + jnp.sum(
        jnp.where((ia0 == 0) & (ia1 == 0) & (ia2 == _N - 1), table, 0.0))


def kernel(encoder_features, word_ids, Wt1, bt1, Wt2, bt2, Ww1, bw1, Ww2, bw2):
    f32 = jnp.float32
    bw2p = jnp.pad(bw2, (0, _NT * _VT - _V),
                   constant_values=_NEG).reshape(_NT, 1, _VT)
    wids = word_ids.astype(f32).reshape(_N, 1)

    full = lambda shp: pl.BlockSpec(shp, lambda t: tuple(0 for _ in shp))
    lr, rp, delta = pl.pallas_call(
        _mlp_kernel,
        grid=(_NT,),
        in_specs=[
            full((_N, _H)),
            full((2 * _H, _H)),
            full((1, _H)),
            full((_H, 1)),
            full((1, 1)),
            full((2 * _H, _H)),
            full((1, _H)),
            full((_N, 1)),
            full((_R, _N)),
            full((_R, _N)),
            full((_R, _N)),
            pl.BlockSpec((_H, _VT), lambda t: (0, t)),
            pl.BlockSpec((1, 1, _VT), lambda t: (t, 0, 0)),
        ],
        out_specs=[full((_R, 1)), full((_R, 1)), full((_R, 1))],
        out_shape=[jax.ShapeDtypeStruct((_R, 1), f32)] * 3,
        scratch_shapes=[
            pltpu.VMEM((_R, _H), f32),
            pltpu.VMEM((_R, 1), f32),
            pltpu.VMEM((_R, 1), f32),
            pltpu.VMEM((_R, 1), f32),
        ],
    )(encoder_features, Wt1, bt1.reshape(1, _H), Wt2, bt2.reshape(1, 1),
      Ww1, bw1.reshape(1, _H), wids,
      jnp.asarray(_GI), jnp.asarray(_GJ), jnp.asarray(_GW), Ww2, bw2p)

    outv = pl.pallas_call(
        _dp_kernel,
        out_shape=jax.ShapeDtypeStruct((1, 1), f32),
    )(lr, rp, delta, jnp.asarray(_GIT), jnp.asarray(_GJ),
      jnp.asarray(_RJT), jnp.asarray(_RQ))
    return outv.reshape(())


# speedup vs baseline: 12.9074x; 1.0449x over previous
import numpy as np
import jax
import jax.numpy as jnp
from jax.experimental import pallas as pl
from jax.experimental.pallas import tpu as pltpu

_N = 32
_P = _N * (_N - 1) // 2
_R = 512
_H = 512
_V = 10000
_VT = 1280
_NT = 8
_NEG = -1.0e30


def _build_consts():
    gi = np.zeros((_R, _N), np.float32)
    gj = np.zeros((_R, _N), np.float32)
    gw = np.zeros((_R, _N), np.float32)
    p = 0
    for i in range(_N):
        for j in range(i + 1, _N):
            gi[p, i] = 1.0
            gj[p, j] = 1.0
            if j <= _N - 2:
                gw[p, j + 1] = 1.0
            p += 1
    gi[_P, 0] = 1.0
    gj[_P, 0] = 1.0
    gw[_P, 1] = 1.0
    rj = np.zeros((_R, _N), np.float32)
    rq = np.zeros((_R, _N), np.float32)
    for j in range(1, _N):
        for q in range(j):
            r = j * (j - 1) // 2 + q
            rj[r, j] = 1.0
            rq[r, q] = 1.0
    return gi, gj, gw, gi.T.copy(), rj.T.copy(), rq


_GI, _GJ, _GW, _GIT, _RJT, _RQ = _build_consts()


def _fused_kernel(enc, wt1, bt1, wt2, bt2, ww1, bw1, wids, gi, gj, gw,
                  git, rjt, rqm, ww2, bw2, out,
                  hw_s, m_s, s_s, tl_s, lr_s, rp_s, wid_s):
    t = pl.program_id(0)

    @pl.when(t == 0)
    def _prologue():
        e = enc[...]
        gim = gi[...]
        gjm = gj[...]
        at = jnp.dot(e, wt1[0:_H, :], preferred_element_type=jnp.float32)
        bt = jnp.dot(e, wt1[_H:2 * _H, :], preferred_element_type=jnp.float32)
        ht = jnp.maximum(
            jnp.dot(gim, at, preferred_element_type=jnp.float32)
            + jnp.dot(gjm, bt, preferred_element_type=jnp.float32)
            + bt1[...], 0.0)
        zt = jnp.dot(ht, wt2[...], preferred_element_type=jnp.float32) + bt2[...]
        sp = jnp.log(1.0 + jnp.exp(-jnp.abs(zt)))
        lr_s[...] = -(jnp.maximum(-zt, 0.0) + sp)
        rp_s[...] = -(jnp.maximum(zt, 0.0) + sp)
        aw = jnp.dot(e, ww1[0:_H, :], preferred_element_type=jnp.float32)
        bw = jnp.dot(e, ww1[_H:2 * _H, :], preferred_element_type=jnp.float32)
        hw_s[...] = jnp.maximum(
            jnp.dot(gim, aw, preferred_element_type=jnp.float32)
            + jnp.dot(gjm, bw, preferred_element_type=jnp.float32)
            + bw1[...], 0.0).astype(jnp.bfloat16)
        wid_s[...] = jnp.dot(gw[...], wids[...],
                             preferred_element_type=jnp.float32)
        m_s[...] = jnp.full((_R, 1), _NEG, jnp.float32)
        s_s[...] = jnp.zeros((_R, 1), jnp.float32)
        tl_s[...] = jnp.zeros((_R, 1), jnp.float32)

    @pl.when(t < _NT)
    def _vocab_tile():
        wcol = jax.lax.broadcasted_iota(jnp.int32, (_R, _VT), 1) + t * _VT
        logits = jnp.dot(hw_s[...], ww2[...].astype(jnp.bfloat16),
                         preferred_element_type=jnp.float32) + bw2[0]
        logits = jnp.where(wcol < _V, logits, _NEG)
        mt = jnp.max(logits, axis=1, keepdims=True)
        mn = jnp.maximum(m_s[...], mt)
        s_s[...] = (s_s[...] * jnp.exp(m_s[...] - mn)
                    + jnp.sum(jnp.exp(logits - mn), axis=1, keepdims=True))
        m_s[...] = mn
        wid = wid_s[...].astype(jnp.int32)
        tl_s[...] = tl_s[...] + jnp.sum(
            jnp.where(wcol == wid, logits, 0.0), axis=1, keepdims=True)

    @pl.when(t == _NT)
    def _dp():
        delta = tl_s[...] - (jnp.log(s_s[...]) + m_s[...])
        gjm = gj[...]
        rpmat = jnp.dot(git[...], rp_s[...] * gjm,
                        preferred_element_type=jnp.float32)
        dpmat = jnp.dot(git[...], delta * gjm,
                        preferred_element_type=jnp.float32)
        lrmat = jnp.dot(rjt[...], lr_s[...] * rqm[...],
                        preferred_element_type=jnp.float32)

        ia0 = jax.lax.broadcasted_iota(jnp.int32, (_N, _N, _N), 0)
        ia1 = jax.lax.broadcasted_iota(jnp.int32, (_N, _N, _N), 1)
        ia2 = jax.lax.broadcasted_iota(jnp.int32, (_N, _N, _N), 2)
        i2 = jax.lax.broadcasted_iota(jnp.int32, (_N, _N), 0)
        p2 = jax.lax.broadcasted_iota(jnp.int32, (_N, _N), 1)

        vmat = dpmat + jnp.where(p2 > i2, rpmat, 0.0)
        maskd = (ia2 == ia1 + 1) & (((ia1 > ia0) & (ia1 <= _N - 2))
                                    | ((ia0 == 0) & (ia1 == 0)))
        table = jnp.where(maskd, vmat[:, :, None], _NEG)

        def body(gap, tbl):
            tj = jnp.sum(jnp.where(ia2 == ia0 + gap, tbl, 0.0), axis=2)
            lrg = jnp.sum(jnp.where(ia1 == ia0 + gap, lrmat[None, :, :], 0.0),
                          axis=1)
            b = tj + lrg
            valid = (p2 >= i2 + 1) & (p2 <= i2 + gap - 1) & (i2 + gap <= _N - 1)
            b = jnp.where(valid, b, _NEG)
            scores = tbl + b[None, :, :]
            mx = jnp.max(scores, axis=2)
            new = jnp.log(jnp.sum(jnp.exp(scores - mx[:, :, None]),
                                  axis=2)) + mx
            wm = (ia2 == ia1 + gap) & ((ia0 < ia1) | ((ia0 == 0) & (ia1 == 0)))
            return jnp.where(wm, new[:, :, None], tbl)

        table = jax.lax.fori_loop(2, _N, body, table)
        out[...] = jnp.full((1, 1), 0.0) + jnp.sum(
            jnp.where((ia0 == 0) & (ia1 == 0) & (ia2 == _N - 1), table, 0.0))


def kernel(encoder_features, word_ids, Wt1, bt1, Wt2, bt2, Ww1, bw1, Ww2, bw2):
    f32 = jnp.float32
    bw2p = jnp.pad(bw2, (0, _NT * _VT - _V),
                   constant_values=_NEG).reshape(_NT, 1, _VT)
    wids = word_ids.astype(f32).reshape(_N, 1)

    full = lambda shp: pl.BlockSpec(shp, lambda t: tuple(0 for _ in shp))
    outv = pl.pallas_call(
        _fused_kernel,
        grid=(_NT + 1,),
        in_specs=[
            full((_N, _H)),
            full((2 * _H, _H)),
            full((1, _H)),
            full((_H, 1)),
            full((1, 1)),
            full((2 * _H, _H)),
            full((1, _H)),
            full((_N, 1)),
            full((_R, _N)),
            full((_R, _N)),
            full((_R, _N)),
            full((_N, _R)),
            full((_N, _R)),
            full((_R, _N)),
            pl.BlockSpec((_H, _VT),
                         lambda t: (0, jnp.minimum(t, _NT - 1))),
            pl.BlockSpec((1, 1, _VT),
                         lambda t: (jnp.minimum(t, _NT - 1), 0, 0)),
        ],
        out_specs=pl.BlockSpec((1, 1), lambda t: (0, 0)),
        out_shape=jax.ShapeDtypeStruct((1, 1), f32),
        scratch_shapes=[
            pltpu.VMEM((_R, _H), jnp.bfloat16),
            pltpu.VMEM((_R, 1), f32),
            pltpu.VMEM((_R, 1), f32),
            pltpu.VMEM((_R, 1), f32),
            pltpu.VMEM((_R, 1), f32),
            pltpu.VMEM((_R, 1), f32),
            pltpu.VMEM((_R, 1), f32),
        ],
    )(encoder_features, Wt1, bt1.reshape(1, _H), Wt2, bt2.reshape(1, 1),
      Ww1, bw1.reshape(1, _H), wids,
      jnp.asarray(_GI), jnp.asarray(_GJ), jnp.asarray(_GW),
      jnp.asarray(_GIT), jnp.asarray(_RJT), jnp.asarray(_RQ), Ww2, bw2p)
    return outv.reshape(())
